# Initial kernel scaffold; baseline (speedup 1.0000x reference)
import jax
import jax.numpy as jnp
from jax.experimental import pallas as pl


def kernel(features, edge_index, W):
    raise NotImplementedError("write your pallas kernel here")


# speedup vs baseline: 4.0253x; 4.0253x over previous
import jax
import jax.numpy as jnp
from jax import lax
from jax.experimental import pallas as pl
from jax.experimental.pallas import tpu as pltpu
from jax.experimental.pallas import tpu_sc as plsc

N_NODES = 10000
N_EDGES = 320000
FEAT = 128

NC = 1
NS = 16
NW = NC * NS
EDGES_PER_W = N_EDGES // NW
CHUNK = 80
BCH = 25
NBLK = EDGES_PER_W // (BCH * CHUNK)
N_PAD = 10240
ROWS_PER_TILE = N_PAD // NS


def _sc_body(feat_hbm, dst_hbm, src_hbm, out_hbm, idx_d, idx_s, rows, ones_v,
             acc, sem):
    c = lax.axis_index("c")
    s = lax.axis_index("s")
    w = c * NS + s

    z16 = jnp.zeros((16,), jnp.float32)
    o16 = jnp.ones((16,), jnp.float32)

    def fill_body(i, _):
        for j in range(FEAT // 16):
            rows[i, pl.ds(j * 16, 16)] = z16
            ones_v[i, pl.ds(j * 16, 16)] = o16
        return 0

    lax.fori_loop(0, CHUNK, fill_body, 0)

    base = s * ROWS_PER_TILE

    def zero_acc():
        for k in range(ROWS_PER_TILE // CHUNK):
            pltpu.sync_copy(rows, acc.at[pl.ds(base + k * CHUNK, CHUNK)])

    zero_acc()
    plsc.subcore_barrier()

    def cblk_body(b, _):
        pltpu.sync_copy(dst_hbm.at[w * NBLK + b], idx_d)

        def ch_body(j, _):
            pltpu.sync_copy(ones_v, acc.at[idx_d.at[j]], add=True)
            return 0

        lax.fori_loop(0, BCH, ch_body, 0)
        return 0

    lax.fori_loop(0, NBLK, cblk_body, 0)

    plsc.subcore_barrier()
    pltpu.sync_copy(acc.at[pl.ds(base, ROWS_PER_TILE)],
                    out_hbm.at[pl.ds(base, ROWS_PER_TILE)])
    zero_acc()
    plsc.subcore_barrier()

    def blk_body(b, _):
        pltpu.sync_copy(dst_hbm.at[w * NBLK + b], idx_d)
        pltpu.sync_copy(src_hbm.at[w * NBLK + b], idx_s)

        def edge_body(j, _):
            pltpu.async_copy(feat_hbm.at[idx_s.at[j]], rows, sem).wait()
            pltpu.sync_copy(rows, acc.at[idx_d.at[j]], add=True)
            return 0

        lax.fori_loop(0, BCH, edge_body, 0)
        return 0

    lax.fori_loop(0, NBLK, blk_body, 0)

    plsc.subcore_barrier()
    pltpu.sync_copy(acc.at[pl.ds(base, ROWS_PER_TILE)],
                    out_hbm.at[pl.ds(N_PAD + base, ROWS_PER_TILE)])


def _sc_aggregate(features, dst, src):
    mesh = plsc.VectorSubcoreMesh(core_axis_name="c", subcore_axis_name="s",
                                  num_cores=NC, num_subcores=NS)
    f = pl.kernel(
        _sc_body,
        out_type=jax.ShapeDtypeStruct((2 * N_PAD, FEAT), jnp.float32),
        mesh=mesh,
        scratch_types=[
            pltpu.VMEM((BCH, CHUNK), jnp.int32),
            pltpu.VMEM((BCH, CHUNK), jnp.int32),
            pltpu.VMEM((CHUNK, FEAT), jnp.float32),
            pltpu.VMEM((CHUNK, FEAT), jnp.float32),
            pltpu.VMEM_SHARED((N_PAD, FEAT), jnp.float32),
            pltpu.SemaphoreType.DMA,
        ],
        name="gc_sc_aggregate",
    )
    return f(features, dst, src)


def _tc_body(f_ref, w_ref, c_ref, s_ref, o_ref):
    w = w_ref[...]
    nodes = jnp.dot(f_ref[...], w, preferred_element_type=jnp.float32)
    counts = c_ref[:, 0:1]
    agg = s_ref[...] / jnp.maximum(counts, 1.0)
    msgs = jnp.dot(agg, w, preferred_element_type=jnp.float32)
    o_ref[:, 0:FEAT] = nodes
    o_ref[:, FEAT:] = msgs


def _tc_finish(features, W, combo):
    blk = 2048
    sum_blk0 = N_PAD // blk
    return pl.pallas_call(
        _tc_body,
        grid=(5,),
        in_specs=[
            pl.BlockSpec((blk, FEAT), lambda i: (i, 0)),
            pl.BlockSpec((FEAT, FEAT), lambda i: (0, 0)),
            pl.BlockSpec((blk, FEAT), lambda i: (i, 0)),
            pl.BlockSpec((blk, FEAT), lambda i: (i + sum_blk0, 0)),
        ],
        out_specs=pl.BlockSpec((blk, 2 * FEAT), lambda i: (i, 0)),
        out_shape=jax.ShapeDtypeStruct((N_NODES, 2 * FEAT), jnp.float32),
    )(features, W, combo, combo)


def kernel(features, edge_index, W):
    dst = edge_index[0].reshape(NW * NBLK, BCH, CHUNK)
    src = edge_index[1].reshape(NW * NBLK, BCH, CHUNK)
    combo = _sc_aggregate(features, dst, src)
    return _tc_finish(features, W, combo)


# speedup vs baseline: 7.0830x; 1.7596x over previous
import jax
import jax.numpy as jnp
from jax import lax
from jax.experimental import pallas as pl
from jax.experimental.pallas import tpu as pltpu
from jax.experimental.pallas import tpu_sc as plsc

N_NODES = 10000
N_EDGES = 320000
FEAT = 128

NC = 2
NS = 16
NW = NC * NS
EDGES_PER_W = N_EDGES // NW
CHUNK = 80
BCH = 25
NBLK = EDGES_PER_W // (BCH * CHUNK)
N_PAD = 10240
ROWS_PER_TILE = N_PAD // NS


def _sc_body(feat_hbm, dst_hbm, src_hbm, out_hbm, idx_d, idx_s, rows, ones_v,
             acc, sem):
    c = lax.axis_index("c")
    s = lax.axis_index("s")
    w = c * NS + s

    z16 = jnp.zeros((16,), jnp.float32)
    o16 = jnp.ones((16,), jnp.float32)

    def fill_body(i, _):
        for j in range(FEAT // 16):
            rows[i, pl.ds(j * 16, 16)] = z16
            ones_v[i, pl.ds(j * 16, 16)] = o16
        return 0

    lax.fori_loop(0, CHUNK, fill_body, 0)

    base = s * ROWS_PER_TILE

    def zero_acc():
        for k in range(ROWS_PER_TILE // CHUNK):
            pltpu.sync_copy(rows, acc.at[pl.ds(base + k * CHUNK, CHUNK)])

    zero_acc()
    plsc.subcore_barrier()

    def cblk_body(b, _):
        pltpu.sync_copy(dst_hbm.at[w * NBLK + b], idx_d)

        def ch_body(j, _):
            pltpu.sync_copy(ones_v, acc.at[idx_d.at[j]], add=True)
            return 0

        lax.fori_loop(0, BCH, ch_body, 0)
        return 0

    lax.fori_loop(0, NBLK, cblk_body, 0)

    plsc.subcore_barrier()
    pltpu.sync_copy(acc.at[pl.ds(base, ROWS_PER_TILE)],
                    out_hbm.at[pl.ds(c * N_PAD + base, ROWS_PER_TILE)])
    zero_acc()
    plsc.subcore_barrier()

    def blk_body(b, _):
        pltpu.sync_copy(dst_hbm.at[w * NBLK + b], idx_d)
        pltpu.sync_copy(src_hbm.at[w * NBLK + b], idx_s)

        def edge_body(j, _):
            pltpu.async_copy(feat_hbm.at[idx_s.at[j]], rows, sem).wait()
            pltpu.sync_copy(rows, acc.at[idx_d.at[j]], add=True)
            return 0

        lax.fori_loop(0, BCH, edge_body, 0)
        return 0

    lax.fori_loop(0, NBLK, blk_body, 0)

    plsc.subcore_barrier()
    pltpu.sync_copy(acc.at[pl.ds(base, ROWS_PER_TILE)],
                    out_hbm.at[pl.ds((2 + c) * N_PAD + base, ROWS_PER_TILE)])


def _sc_aggregate(features, dst, src):
    mesh = plsc.VectorSubcoreMesh(core_axis_name="c", subcore_axis_name="s",
                                  num_cores=NC, num_subcores=NS)
    f = pl.kernel(
        _sc_body,
        out_type=jax.ShapeDtypeStruct((4 * N_PAD, FEAT), jnp.float32),
        mesh=mesh,
        scratch_types=[
            pltpu.VMEM((BCH, CHUNK), jnp.int32),
            pltpu.VMEM((BCH, CHUNK), jnp.int32),
            pltpu.VMEM((CHUNK, FEAT), jnp.float32),
            pltpu.VMEM((CHUNK, FEAT), jnp.float32),
            pltpu.VMEM_SHARED((N_PAD, FEAT), jnp.float32),
            pltpu.SemaphoreType.DMA,
        ],
        name="gc_sc_aggregate",
    )
    return f(features, dst, src)


def _tc_body(f_ref, w_ref, c0_ref, c1_ref, s0_ref, s1_ref, o_ref):
    w = w_ref[...]
    nodes = jnp.dot(f_ref[...], w, preferred_element_type=jnp.float32)
    counts = c0_ref[:, 0:1] + c1_ref[:, 0:1]
    agg = (s0_ref[...] + s1_ref[...]) / jnp.maximum(counts, 1.0)
    msgs = jnp.dot(agg, w, preferred_element_type=jnp.float32)
    o_ref[:, 0:FEAT] = nodes
    o_ref[:, FEAT:] = msgs


def _tc_finish(features, W, combo):
    blk = 2048
    pb = N_PAD // blk
    return pl.pallas_call(
        _tc_body,
        grid=(5,),
        in_specs=[
            pl.BlockSpec((blk, FEAT), lambda i: (i, 0)),
            pl.BlockSpec((FEAT, FEAT), lambda i: (0, 0)),
            pl.BlockSpec((blk, FEAT), lambda i: (i, 0)),
            pl.BlockSpec((blk, FEAT), lambda i: (i + pb, 0)),
            pl.BlockSpec((blk, FEAT), lambda i: (i + 2 * pb, 0)),
            pl.BlockSpec((blk, FEAT), lambda i: (i + 3 * pb, 0)),
        ],
        out_specs=pl.BlockSpec((blk, 2 * FEAT), lambda i: (i, 0)),
        out_shape=jax.ShapeDtypeStruct((N_NODES, 2 * FEAT), jnp.float32),
    )(features, W, combo, combo, combo, combo)


def kernel(features, edge_index, W):
    dst = edge_index[0].reshape(NW * NBLK, BCH, CHUNK)
    src = edge_index[1].reshape(NW * NBLK, BCH, CHUNK)
    combo = _sc_aggregate(features, dst, src)
    return _tc_finish(features, W, combo)


# speedup vs baseline: 8.3373x; 1.1771x over previous
import jax
import jax.numpy as jnp
from jax import lax
from jax.experimental import pallas as pl
from jax.experimental.pallas import tpu as pltpu
from jax.experimental.pallas import tpu_sc as plsc

N_NODES = 10000
N_EDGES = 320000
FEAT = 128

NC = 2
NS = 16
NW = NC * NS
EDGES_PER_W = N_EDGES // NW
CHUNK = 80
BCH = 25
NBLK = EDGES_PER_W // (BCH * CHUNK)
N_PAD = 10240
ROWS_PER_TILE = N_PAD // NS


def _sc_body(feat_hbm, dst_hbm, src_hbm, out_hbm, idx_d, idx_s, rows, rows_b,
             ones_v, acc, sem, sem_s):
    c = lax.axis_index("c")
    s = lax.axis_index("s")
    w = c * NS + s

    z16 = jnp.zeros((16,), jnp.float32)
    o16 = jnp.ones((16,), jnp.float32)

    def fill_body(i, _):
        for j in range(FEAT // 16):
            rows[i, pl.ds(j * 16, 16)] = z16
            ones_v[i, pl.ds(j * 16, 16)] = o16
        return 0

    lax.fori_loop(0, CHUNK, fill_body, 0)

    base = s * ROWS_PER_TILE

    def zero_acc():
        for k in range(ROWS_PER_TILE // CHUNK):
            pltpu.sync_copy(rows, acc.at[pl.ds(base + k * CHUNK, CHUNK)])

    zero_acc()
    plsc.subcore_barrier()

    def cblk_body(b, _):
        pltpu.sync_copy(dst_hbm.at[w * NBLK + b], idx_d)
        pend = []
        for j in range(BCH):
            pend.append(pltpu.async_copy(ones_v, acc.at[idx_d.at[j]],
                                         sem_s, add=True))
            if len(pend) >= 3:
                pend.pop(0).wait()
        for cp in pend:
            cp.wait()
        return 0

    lax.fori_loop(0, NBLK, cblk_body, 0)

    plsc.subcore_barrier()
    pltpu.sync_copy(acc.at[pl.ds(base, ROWS_PER_TILE)],
                    out_hbm.at[pl.ds(c * N_PAD + base, ROWS_PER_TILE)])
    zero_acc()
    plsc.subcore_barrier()

    bufs = (rows, rows_b)

    def blk_body(b, _):
        pltpu.sync_copy(dst_hbm.at[w * NBLK + b], idx_d)
        pltpu.sync_copy(src_hbm.at[w * NBLK + b], idx_s)
        gat = pltpu.async_copy(feat_hbm.at[idx_s.at[0]], bufs[0], sem)
        scat = None
        for j in range(BCH):
            cur = bufs[j % 2]
            nxt = bufs[(j + 1) % 2]
            gat.wait()
            if scat is not None:
                scat.wait()
            if j + 1 < BCH:
                gat = pltpu.async_copy(feat_hbm.at[idx_s.at[j + 1]], nxt, sem)
            scat = pltpu.async_copy(cur, acc.at[idx_d.at[j]], sem_s, add=True)
        scat.wait()
        return 0

    lax.fori_loop(0, NBLK, blk_body, 0)

    plsc.subcore_barrier()
    pltpu.sync_copy(acc.at[pl.ds(base, ROWS_PER_TILE)],
                    out_hbm.at[pl.ds((2 + c) * N_PAD + base, ROWS_PER_TILE)])


def _sc_aggregate(features, dst, src):
    mesh = plsc.VectorSubcoreMesh(core_axis_name="c", subcore_axis_name="s",
                                  num_cores=NC, num_subcores=NS)
    f = pl.kernel(
        _sc_body,
        out_type=jax.ShapeDtypeStruct((4 * N_PAD, FEAT), jnp.float32),
        mesh=mesh,
        scratch_types=[
            pltpu.VMEM((BCH, CHUNK), jnp.int32),
            pltpu.VMEM((BCH, CHUNK), jnp.int32),
            pltpu.VMEM((CHUNK, FEAT), jnp.float32),
            pltpu.VMEM((CHUNK, FEAT), jnp.float32),
            pltpu.VMEM((CHUNK, FEAT), jnp.float32),
            pltpu.VMEM_SHARED((N_PAD, FEAT), jnp.float32),
            pltpu.SemaphoreType.DMA,
            pltpu.SemaphoreType.DMA,
        ],
        name="gc_sc_aggregate",
    )
    return f(features, dst, src)


def _tc_body(f_ref, w_ref, c0_ref, c1_ref, s0_ref, s1_ref, o_ref):
    w = w_ref[...]
    nodes = jnp.dot(f_ref[...], w, preferred_element_type=jnp.float32)
    counts = c0_ref[:, 0:1] + c1_ref[:, 0:1]
    agg = (s0_ref[...] + s1_ref[...]) / jnp.maximum(counts, 1.0)
    msgs = jnp.dot(agg, w, preferred_element_type=jnp.float32)
    o_ref[:, 0:FEAT] = nodes
    o_ref[:, FEAT:] = msgs


def _tc_finish(features, W, combo):
    blk = 2048
    pb = N_PAD // blk
    return pl.pallas_call(
        _tc_body,
        grid=(5,),
        in_specs=[
            pl.BlockSpec((blk, FEAT), lambda i: (i, 0)),
            pl.BlockSpec((FEAT, FEAT), lambda i: (0, 0)),
            pl.BlockSpec((blk, FEAT), lambda i: (i, 0)),
            pl.BlockSpec((blk, FEAT), lambda i: (i + pb, 0)),
            pl.BlockSpec((blk, FEAT), lambda i: (i + 2 * pb, 0)),
            pl.BlockSpec((blk, FEAT), lambda i: (i + 3 * pb, 0)),
        ],
        out_specs=pl.BlockSpec((blk, 2 * FEAT), lambda i: (i, 0)),
        out_shape=jax.ShapeDtypeStruct((N_NODES, 2 * FEAT), jnp.float32),
    )(features, W, combo, combo, combo, combo)


def kernel(features, edge_index, W):
    dst = edge_index[0].reshape(NW * NBLK, BCH, CHUNK)
    src = edge_index[1].reshape(NW * NBLK, BCH, CHUNK)
    combo = _sc_aggregate(features, dst, src)
    return _tc_finish(features, W, combo)


# speedup vs baseline: 10.4088x; 1.2485x over previous
import jax
import jax.numpy as jnp
from jax import lax
from jax.experimental import pallas as pl
from jax.experimental.pallas import tpu as pltpu
from jax.experimental.pallas import tpu_sc as plsc

N_NODES = 10000
N_EDGES = 320000
FEAT = 128

NC = 2
NS = 16
NW = NC * NS
EDGES_PER_W = N_EDGES // NW
CHUNK = 80
BCH = 25
NBLK = EDGES_PER_W // (BCH * CHUNK)
N_PAD = 10240
ROWS_PER_TILE = N_PAD // NS


def _sc_body(feat_hbm, dst_hbm, src_hbm, out_hbm, idx_d, idx_s, rows, rows_b,
             ones_v, acc, sem, sem_s):
    c = lax.axis_index("c")
    s = lax.axis_index("s")
    w = c * NS + s

    z16 = jnp.zeros((16,), jnp.float32)
    o16 = jnp.ones((16,), jnp.float32)

    def fill_body(i, _):
        for j in range(FEAT // 16):
            rows[i, pl.ds(j * 16, 16)] = z16
            ones_v[i, pl.ds(j * 16, 16)] = o16
        return 0

    lax.fori_loop(0, CHUNK, fill_body, 0)

    base = s * ROWS_PER_TILE

    def zero_acc():
        for k in range(ROWS_PER_TILE // CHUNK):
            pltpu.sync_copy(rows, acc.at[pl.ds(base + k * CHUNK, CHUNK)])

    zero_acc()
    plsc.subcore_barrier()

    def cblk_body(b, _):
        pltpu.sync_copy(dst_hbm.at[w * NBLK + b], idx_d)
        pend = []
        for j in range(BCH):
            pend.append(pltpu.async_copy(ones_v, acc.at[idx_d.at[j]],
                                         sem_s, add=True))
            if len(pend) >= 5:
                pend.pop(0).wait()
        for cp in pend:
            cp.wait()
        return 0

    lax.fori_loop(0, NBLK, cblk_body, 0)

    plsc.subcore_barrier()
    pltpu.sync_copy(acc.at[pl.ds(base, ROWS_PER_TILE)],
                    out_hbm.at[pl.ds(c * N_PAD + base, ROWS_PER_TILE)])
    zero_acc()
    plsc.subcore_barrier()

    bufs = (rows, rows_b, ones_v)

    def blk_body(b, _):
        pltpu.sync_copy(dst_hbm.at[w * NBLK + b], idx_d)
        pltpu.sync_copy(src_hbm.at[w * NBLK + b], idx_s)
        gats = [pltpu.async_copy(feat_hbm.at[idx_s.at[0]], bufs[0], sem),
                pltpu.async_copy(feat_hbm.at[idx_s.at[1]], bufs[1], sem)]
        scats = [None, None]
        for j in range(BCH):
            gats[j % 2].wait()
            prev = scats[(j + 1) % 2]
            if prev is not None:
                prev.wait()
            if j + 2 < BCH:
                gats[j % 2] = pltpu.async_copy(
                    feat_hbm.at[idx_s.at[j + 2]], bufs[(j + 2) % 3], sem)
            scats[j % 2] = pltpu.async_copy(bufs[j % 3], acc.at[idx_d.at[j]],
                                            sem_s, add=True)
        scats[(BCH - 1) % 2].wait()
        return 0

    lax.fori_loop(0, NBLK, blk_body, 0)

    plsc.subcore_barrier()
    pltpu.sync_copy(acc.at[pl.ds(base, ROWS_PER_TILE)],
                    out_hbm.at[pl.ds((2 + c) * N_PAD + base, ROWS_PER_TILE)])


def _sc_aggregate(features, dst, src):
    mesh = plsc.VectorSubcoreMesh(core_axis_name="c", subcore_axis_name="s",
                                  num_cores=NC, num_subcores=NS)
    f = pl.kernel(
        _sc_body,
        out_type=jax.ShapeDtypeStruct((4 * N_PAD, FEAT), jnp.float32),
        mesh=mesh,
        scratch_types=[
            pltpu.VMEM((BCH, CHUNK), jnp.int32),
            pltpu.VMEM((BCH, CHUNK), jnp.int32),
            pltpu.VMEM((CHUNK, FEAT), jnp.float32),
            pltpu.VMEM((CHUNK, FEAT), jnp.float32),
            pltpu.VMEM((CHUNK, FEAT), jnp.float32),
            pltpu.VMEM_SHARED((N_PAD, FEAT), jnp.float32),
            pltpu.SemaphoreType.DMA,
            pltpu.SemaphoreType.DMA,
        ],
        name="gc_sc_aggregate",
    )
    return f(features, dst, src)


def _tc_nodes_body(f_ref, w_ref, o_ref):
    o_ref[...] = jnp.dot(f_ref[...], w_ref[...],
                         preferred_element_type=jnp.float32)


def _tc_nodes(features, W):
    blk = 2000
    return pl.pallas_call(
        _tc_nodes_body,
        grid=(N_NODES // blk,),
        in_specs=[
            pl.BlockSpec((blk, FEAT), lambda i: (i, 0)),
            pl.BlockSpec((FEAT, FEAT), lambda i: (0, 0)),
        ],
        out_specs=pl.BlockSpec((blk, FEAT), lambda i: (i, 0)),
        out_shape=jax.ShapeDtypeStruct((N_NODES, FEAT), jnp.float32),
    )(features, W)


def _tc_body(n_ref, w_ref, c0_ref, c1_ref, s0_ref, s1_ref, o_ref):
    counts = c0_ref[:, 0:1] + c1_ref[:, 0:1]
    agg = (s0_ref[...] + s1_ref[...]) / jnp.maximum(counts, 1.0)
    msgs = jnp.dot(agg, w_ref[...], preferred_element_type=jnp.float32)
    o_ref[:, 0:FEAT] = n_ref[...]
    o_ref[:, FEAT:] = msgs


def _tc_finish(nodes, W, combo):
    blk = 2048
    pb = N_PAD // blk
    return pl.pallas_call(
        _tc_body,
        grid=(5,),
        in_specs=[
            pl.BlockSpec((blk, FEAT), lambda i: (i, 0)),
            pl.BlockSpec((FEAT, FEAT), lambda i: (0, 0)),
            pl.BlockSpec((blk, FEAT), lambda i: (i, 0)),
            pl.BlockSpec((blk, FEAT), lambda i: (i + pb, 0)),
            pl.BlockSpec((blk, FEAT), lambda i: (i + 2 * pb, 0)),
            pl.BlockSpec((blk, FEAT), lambda i: (i + 3 * pb, 0)),
        ],
        out_specs=pl.BlockSpec((blk, 2 * FEAT), lambda i: (i, 0)),
        out_shape=jax.ShapeDtypeStruct((N_NODES, 2 * FEAT), jnp.float32),
    )(nodes, W, combo, combo, combo, combo)


def kernel(features, edge_index, W):
    dst = edge_index[0].reshape(NW * NBLK, BCH, CHUNK)
    src = edge_index[1].reshape(NW * NBLK, BCH, CHUNK)
    combo = _sc_aggregate(features, dst, src)
    nodes = _tc_nodes(features, W)
    return _tc_finish(nodes, W, combo)


# speedup vs baseline: 10.7895x; 1.0366x over previous
import jax
import jax.numpy as jnp
from jax import lax
from jax.experimental import pallas as pl
from jax.experimental.pallas import tpu as pltpu
from jax.experimental.pallas import tpu_sc as plsc

N_NODES = 10000
N_EDGES = 320000
FEAT = 128

NC = 2
NS = 16
NW = NC * NS
EDGES_PER_W = N_EDGES // NW
CHUNK = 80
BCH = 25
NBLK = EDGES_PER_W // (BCH * CHUNK)
N_PAD = 10240
ROWS_PER_TILE = N_PAD // NS


def _sc_body(feat_hbm, dst_hbm, src_hbm, out_hbm, idx_d, idx_s, rows, rows_b,
             ones_v, acc, sem, sem_s, sem_i):
    c = lax.axis_index("c")
    s = lax.axis_index("s")
    w = c * NS + s

    z16 = jnp.zeros((16,), jnp.float32)
    o16 = jnp.ones((16,), jnp.float32)

    def fill_body(i, _):
        for j in range(FEAT // 16):
            rows[i, pl.ds(j * 16, 16)] = z16
            ones_v[i, pl.ds(j * 16, 16)] = o16
        return 0

    lax.fori_loop(0, CHUNK, fill_body, 0)

    base = s * ROWS_PER_TILE

    def zero_acc():
        zs = [pltpu.async_copy(rows, acc.at[pl.ds(base + k * CHUNK, CHUNK)],
                               sem_s)
              for k in range(ROWS_PER_TILE // CHUNK)]
        for z in zs:
            z.wait()

    zero_acc()
    plsc.subcore_barrier()

    pltpu.sync_copy(dst_hbm.at[w * NBLK], idx_d.at[0])

    def cblk_body(b, _):
        p = lax.rem(b, 2)
        bn = jnp.minimum(b + 1, NBLK - 1)
        pre = pltpu.async_copy(dst_hbm.at[w * NBLK + bn], idx_d.at[1 - p], sem)
        pend = []
        for j in range(BCH):
            pend.append(pltpu.async_copy(ones_v, acc.at[idx_d.at[p, j]],
                                         sem_s, add=True))
            if len(pend) >= 4:
                pend.pop(0).wait()
        for cp in pend:
            cp.wait()
        pre.wait()
        return 0

    lax.fori_loop(0, NBLK, cblk_body, 0)

    plsc.subcore_barrier()
    pltpu.sync_copy(acc.at[pl.ds(base, ROWS_PER_TILE)],
                    out_hbm.at[pl.ds(c * N_PAD + base, ROWS_PER_TILE)])
    zero_acc()
    plsc.subcore_barrier()

    bufs = (rows, rows_b, ones_v)

    pltpu.sync_copy(dst_hbm.at[w * NBLK], idx_d.at[0])
    pltpu.sync_copy(src_hbm.at[w * NBLK], idx_s.at[0])

    def blk_body(b, _):
        p = lax.rem(b, 2)
        bn = jnp.minimum(b + 1, NBLK - 1)
        pre_d = pltpu.async_copy(dst_hbm.at[w * NBLK + bn], idx_d.at[1 - p],
                                 sem_i)
        pre_s = pltpu.async_copy(src_hbm.at[w * NBLK + bn], idx_s.at[1 - p],
                                 sem_i)
        gats = [pltpu.async_copy(feat_hbm.at[idx_s.at[p, 0]], bufs[0], sem),
                pltpu.async_copy(feat_hbm.at[idx_s.at[p, 1]], bufs[1], sem)]
        scats = [None, None]
        for j in range(BCH):
            gats[j % 2].wait()
            prev = scats[(j + 1) % 2]
            if prev is not None:
                prev.wait()
            if j + 2 < BCH:
                gats[j % 2] = pltpu.async_copy(
                    feat_hbm.at[idx_s.at[p, j + 2]], bufs[(j + 2) % 3], sem)
            scats[j % 2] = pltpu.async_copy(bufs[j % 3],
                                            acc.at[idx_d.at[p, j]],
                                            sem_s, add=True)
        scats[(BCH - 1) % 2].wait()
        pre_d.wait()
        pre_s.wait()
        return 0

    lax.fori_loop(0, NBLK, blk_body, 0)

    plsc.subcore_barrier()
    pltpu.sync_copy(acc.at[pl.ds(base, ROWS_PER_TILE)],
                    out_hbm.at[pl.ds((2 + c) * N_PAD + base, ROWS_PER_TILE)])


def _sc_aggregate(features, dst, src):
    mesh = plsc.VectorSubcoreMesh(core_axis_name="c", subcore_axis_name="s",
                                  num_cores=NC, num_subcores=NS)
    f = pl.kernel(
        _sc_body,
        out_type=jax.ShapeDtypeStruct((4 * N_PAD, FEAT), jnp.float32),
        mesh=mesh,
        scratch_types=[
            pltpu.VMEM((2, BCH, CHUNK), jnp.int32),
            pltpu.VMEM((2, BCH, CHUNK), jnp.int32),
            pltpu.VMEM((CHUNK, FEAT), jnp.float32),
            pltpu.VMEM((CHUNK, FEAT), jnp.float32),
            pltpu.VMEM((CHUNK, FEAT), jnp.float32),
            pltpu.VMEM_SHARED((N_PAD, FEAT), jnp.float32),
            pltpu.SemaphoreType.DMA,
            pltpu.SemaphoreType.DMA,
            pltpu.SemaphoreType.DMA,
        ],
        name="gc_sc_aggregate",
    )
    return f(features, dst, src)


def _tc_body(f_ref, w_ref, c0_ref, c1_ref, s0_ref, s1_ref, o_ref):
    w = w_ref[...]
    nodes = jnp.dot(f_ref[...], w, preferred_element_type=jnp.float32)
    counts = c0_ref[:, 0:1] + c1_ref[:, 0:1]
    agg = (s0_ref[...] + s1_ref[...]) / jnp.maximum(counts, 1.0)
    msgs = jnp.dot(agg, w, preferred_element_type=jnp.float32)
    o_ref[:, 0:FEAT] = nodes
    o_ref[:, FEAT:] = msgs


def _tc_finish(features, W, combo):
    blk = 2048
    pb = N_PAD // blk
    return pl.pallas_call(
        _tc_body,
        grid=(5,),
        in_specs=[
            pl.BlockSpec((blk, FEAT), lambda i: (i, 0)),
            pl.BlockSpec((FEAT, FEAT), lambda i: (0, 0)),
            pl.BlockSpec((blk, FEAT), lambda i: (i, 0)),
            pl.BlockSpec((blk, FEAT), lambda i: (i + pb, 0)),
            pl.BlockSpec((blk, FEAT), lambda i: (i + 2 * pb, 0)),
            pl.BlockSpec((blk, FEAT), lambda i: (i + 3 * pb, 0)),
        ],
        out_specs=pl.BlockSpec((blk, 2 * FEAT), lambda i: (i, 0)),
        out_shape=jax.ShapeDtypeStruct((N_NODES, 2 * FEAT), jnp.float32),
    )(features, W, combo, combo, combo, combo)


def kernel(features, edge_index, W):
    dst = edge_index[0].reshape(NW * NBLK, BCH, CHUNK)
    src = edge_index[1].reshape(NW * NBLK, BCH, CHUNK)
    combo = _sc_aggregate(features, dst, src)
    return _tc_finish(features, W, combo)
